# baseline (speedup 1.0000x reference)
import functools
import math

import numpy as np
import jax
import jax.numpy as jnp
from jax import lax
from jax.experimental import pallas as pl
from jax.experimental.pallas import tpu as pltpu
from jax.experimental.pallas import tpu_sc as plsc

D_MODEL = 512
SEQ_LEN = 8192
SCALE = math.sqrt(D_MODEL)

NUM_CORES = 2
NUM_SUBCORES = 16
NUM_WORKERS = NUM_CORES * NUM_SUBCORES
ROWS_PER_WORKER = SEQ_LEN // NUM_WORKERS
CHUNK = 32
NUM_CHUNKS = ROWS_PER_WORKER // CHUNK
LANES = 16


def _host_tables():
    half = np.arange(0, D_MODEL, 2, dtype=np.float32).astype(np.float64)
    w = np.exp(half * (-math.log(10000.0) / D_MODEL))

    def pe_row(s):
        row = np.empty((D_MODEL,), dtype=np.float64)
        row[0::2] = np.sin(s * w)
        row[1::2] = np.cos(s * w)
        return row

    b = np.stack([pe_row(k * ROWS_PER_WORKER) for k in range(NUM_WORKERS)])
    bswap = np.empty_like(b)
    bswap[:, 0::2] = b[:, 1::2]
    bswap[:, 1::2] = b[:, 0::2]

    r = np.arange(ROWS_PER_WORKER, dtype=np.float64)[:, None]
    cosr = np.cos(r * w[None, :])
    sinr = np.sin(r * w[None, :])
    xq = np.empty((ROWS_PER_WORKER, D_MODEL), dtype=np.float32)
    yq = np.empty((ROWS_PER_WORKER, D_MODEL), dtype=np.float32)
    xq[:, 0::2] = cosr
    xq[:, 1::2] = cosr
    yq[:, 0::2] = sinr
    yq[:, 1::2] = -sinr

    def bf16_bits(x):
        u = x.astype(np.float32).view(np.uint32)
        rnd = ((u >> 16) & 1) + 0x7FFF
        return ((u + rnd) >> 16).astype(np.uint32)

    packed = (bf16_bits(yq) << 16) | bf16_bits(xq)
    return (
        packed.astype(np.int32),
        b.astype(np.float32),
        bswap.astype(np.float32),
    )


_XYQ, _B, _BSWAP = _host_tables()

_MESH = plsc.VectorSubcoreMesh(core_axis_name="c", subcore_axis_name="s")


@functools.partial(
    pl.kernel,
    mesh=_MESH,
    out_type=jax.ShapeDtypeStruct((SEQ_LEN, D_MODEL), jnp.float32),
    scratch_types=[
        pltpu.VMEM((ROWS_PER_WORKER,), jnp.int32),
        pltpu.VMEM((2, CHUNK, D_MODEL), jnp.float32),
        pltpu.VMEM((2, CHUNK, D_MODEL), jnp.float32),
        pltpu.VMEM((2, CHUNK, D_MODEL), jnp.int32),
        pltpu.VMEM((NUM_SUBCORES, D_MODEL), jnp.int32),
        pltpu.VMEM((1, D_MODEL), jnp.float32),
        pltpu.VMEM((1, D_MODEL), jnp.float32),
        pltpu.VMEM_SHARED((ROWS_PER_WORKER, D_MODEL), jnp.int32),
        pltpu.SemaphoreType.DMA,
        pltpu.SemaphoreType.DMA,
        pltpu.SemaphoreType.DMA,
        pltpu.SemaphoreType.DMA,
        pltpu.SemaphoreType.DMA,
        pltpu.SemaphoreType.DMA,
    ],
)
def _sc_embed_pe(idx_hbm, xyq_hbm, b_hbm, bswap_hbm, table_hbm, out_hbm,
                 idx_v, rows_v, out_v, xy_v, stage_v, b_v, bs_v, xyq_sh,
                 g_sem0, g_sem1, q_sem0, q_sem1, o_sem0, o_sem1):
    cid = lax.axis_index("c")
    sid = lax.axis_index("s")
    wid = sid * NUM_CORES + cid
    base = wid * ROWS_PER_WORKER

    g_sems = (g_sem0, g_sem1)
    q_sems = (q_sem0, q_sem1)
    o_sems = (o_sem0, o_sem1)

    pltpu.sync_copy(xyq_hbm.at[pl.ds(sid * NUM_SUBCORES, NUM_SUBCORES)], stage_v)
    pltpu.sync_copy(stage_v, xyq_sh.at[pl.ds(sid * NUM_SUBCORES, NUM_SUBCORES)])

    pltpu.sync_copy(idx_hbm.at[pl.ds(base, ROWS_PER_WORKER)], idx_v)
    pltpu.sync_copy(b_hbm.at[pl.ds(wid, 1)], b_v)
    pltpu.sync_copy(bswap_hbm.at[pl.ds(wid, 1)], bs_v)

    plsc.subcore_barrier()

    def start_in(g):
        b = g % 2
        gather = pltpu.async_copy(
            table_hbm.at[idx_v.at[pl.ds(g * CHUNK, CHUNK)]],
            rows_v.at[b], g_sems[b])
        xy_cp = pltpu.async_copy(
            xyq_sh.at[pl.ds(g * CHUNK, CHUNK)], xy_v.at[b], q_sems[b])
        return gather, xy_cp

    in_flight = {0: start_in(0), 1: start_in(1)}
    out_flight = {}

    for g in range(NUM_CHUNKS):
        b = g % 2
        if g >= 2:
            out_flight.pop(g - 2).wait()
        gather, xy_cp = in_flight.pop(g)
        gather.wait()
        xy_cp.wait()

        def col_body(c):
            sl = pl.ds(c * LANES, LANES)
            bvec = b_v[0, sl]
            bsvec = bs_v[0, sl]
            for r in range(CHUNK):
                w = xy_v[b, r, sl]
                xq = lax.bitcast_convert_type(w << 16, jnp.float32)
                yq = lax.bitcast_convert_type(w, jnp.float32)
                rv = rows_v[b, r, sl]
                out_v[b, r, sl] = rv * SCALE + (bvec * xq + bsvec * yq)

        lax.fori_loop(0, D_MODEL // LANES,
                      lambda c, _: (col_body(c), 0)[1], 0)

        out_flight[g] = pltpu.async_copy(
            out_v.at[b], out_hbm.at[pl.ds(base + g * CHUNK, CHUNK)],
            o_sems[b])
        if g + 2 < NUM_CHUNKS:
            in_flight[g + 2] = start_in(g + 2)

    out_flight.pop(NUM_CHUNKS - 2).wait()
    out_flight.pop(NUM_CHUNKS - 1).wait()


@jax.jit
def _run(inputs, table):
    xyq = jnp.asarray(_XYQ)
    b = jnp.asarray(_B)
    bswap = jnp.asarray(_BSWAP)
    return _sc_embed_pe(inputs.astype(jnp.int32), xyq, b, bswap, table)


def kernel(inputs, table):
    return _run(inputs, table)


# speedup vs baseline: 1.2234x; 1.2234x over previous
import functools
import math

import numpy as np
import jax
import jax.numpy as jnp
from jax import lax
from jax.experimental import pallas as pl
from jax.experimental.pallas import tpu as pltpu
from jax.experimental.pallas import tpu_sc as plsc

D_MODEL = 512
SEQ_LEN = 8192
SCALE = math.sqrt(D_MODEL)

NUM_CORES = 2
NUM_SUBCORES = 16
NUM_WORKERS = NUM_CORES * NUM_SUBCORES
ROWS_PER_WORKER = SEQ_LEN // NUM_WORKERS
CHUNK = 32
NUM_CHUNKS = ROWS_PER_WORKER // CHUNK
LANES = 16


def _host_tables():
    half = np.arange(0, D_MODEL, 2, dtype=np.float32).astype(np.float64)
    w = np.exp(half * (-math.log(10000.0) / D_MODEL))

    def pe_row(s):
        row = np.empty((D_MODEL,), dtype=np.float64)
        row[0::2] = np.sin(s * w)
        row[1::2] = np.cos(s * w)
        return row

    b = np.stack([pe_row(k * ROWS_PER_WORKER) for k in range(NUM_WORKERS)])
    bswap = np.empty_like(b)
    bswap[:, 0::2] = b[:, 1::2]
    bswap[:, 1::2] = b[:, 0::2]

    r = np.arange(ROWS_PER_WORKER, dtype=np.float64)[:, None]
    cosr = np.cos(r * w[None, :])
    sinr = np.sin(r * w[None, :])
    xq = np.empty((ROWS_PER_WORKER, D_MODEL), dtype=np.float32)
    yq = np.empty((ROWS_PER_WORKER, D_MODEL), dtype=np.float32)
    xq[:, 0::2] = cosr
    xq[:, 1::2] = cosr
    yq[:, 0::2] = sinr
    yq[:, 1::2] = -sinr

    def bf16_bits(x):
        u = x.astype(np.float32).view(np.uint32)
        rnd = ((u >> 16) & 1) + 0x7FFF
        return ((u + rnd) >> 16).astype(np.uint32)

    packed = (bf16_bits(yq) << 16) | bf16_bits(xq)
    return (
        packed.astype(np.int32),
        b.astype(np.float32),
        bswap.astype(np.float32),
    )


_XYQ, _B, _BSWAP = _host_tables()

_MESH = plsc.VectorSubcoreMesh(core_axis_name="c", subcore_axis_name="s")


@functools.partial(
    pl.kernel,
    mesh=_MESH,
    out_type=jax.ShapeDtypeStruct((SEQ_LEN, D_MODEL), jnp.float32),
    scratch_types=[
        pltpu.VMEM((ROWS_PER_WORKER,), jnp.int32),
        pltpu.VMEM((2, CHUNK, D_MODEL), jnp.float32),
        pltpu.VMEM((2, CHUNK, D_MODEL), jnp.float32),
        pltpu.VMEM((2, CHUNK, D_MODEL), jnp.int32),
        pltpu.VMEM((NUM_SUBCORES, D_MODEL), jnp.int32),
        pltpu.VMEM((1, D_MODEL), jnp.float32),
        pltpu.VMEM((1, D_MODEL), jnp.float32),
        pltpu.VMEM_SHARED((ROWS_PER_WORKER, D_MODEL), jnp.int32),
        pltpu.SemaphoreType.DMA,
        pltpu.SemaphoreType.DMA,
        pltpu.SemaphoreType.DMA,
        pltpu.SemaphoreType.DMA,
        pltpu.SemaphoreType.DMA,
        pltpu.SemaphoreType.DMA,
    ],
)
def _sc_embed_pe(idx_hbm, xyq_hbm, b_hbm, bswap_hbm, table_hbm, out_hbm,
                 idx_v, rows_v, out_v, xy_v, stage_v, b_v, bs_v, xyq_sh,
                 g_sem0, g_sem1, q_sem0, q_sem1, o_sem0, o_sem1):
    cid = lax.axis_index("c")
    sid = lax.axis_index("s")
    wid = sid * NUM_CORES + cid
    base = wid * ROWS_PER_WORKER

    g_sems = (g_sem0, g_sem1)
    q_sems = (q_sem0, q_sem1)
    o_sems = (o_sem0, o_sem1)

    pltpu.sync_copy(xyq_hbm.at[pl.ds(sid * NUM_SUBCORES, NUM_SUBCORES)], stage_v)
    pltpu.sync_copy(stage_v, xyq_sh.at[pl.ds(sid * NUM_SUBCORES, NUM_SUBCORES)])

    pltpu.sync_copy(idx_hbm.at[pl.ds(base, ROWS_PER_WORKER)], idx_v)
    pltpu.sync_copy(b_hbm.at[pl.ds(wid, 1)], b_v)
    pltpu.sync_copy(bswap_hbm.at[pl.ds(wid, 1)], bs_v)

    plsc.subcore_barrier()

    def start_in(g):
        b = g % 2
        gather = pltpu.async_copy(
            table_hbm.at[idx_v.at[pl.ds(g * CHUNK, CHUNK)]],
            rows_v.at[b], g_sems[b])
        xy_cp = pltpu.async_copy(
            xyq_sh.at[pl.ds(g * CHUNK, CHUNK)], xy_v.at[b], q_sems[b])
        return gather, xy_cp

    in_flight = {0: start_in(0), 1: start_in(1)}
    out_flight = {}

    for g in range(NUM_CHUNKS):
        b = g % 2
        if g >= 2:
            out_flight.pop(g - 2).wait()
        gather, xy_cp = in_flight.pop(g)
        gather.wait()
        xy_cp.wait()

        @plsc.parallel_loop(0, D_MODEL // LANES, unroll=2)
        def col_body(c):
            sl = pl.ds(c * LANES, LANES)
            bvec = b_v[0, sl]
            bsvec = bs_v[0, sl]
            for r in range(CHUNK):
                w = xy_v[b, r, sl]
                xq = lax.bitcast_convert_type(w << 16, jnp.float32)
                yq = lax.bitcast_convert_type(w, jnp.float32)
                rv = rows_v[b, r, sl]
                out_v[b, r, sl] = rv * SCALE + (bvec * xq + bsvec * yq)

        out_flight[g] = pltpu.async_copy(
            out_v.at[b], out_hbm.at[pl.ds(base + g * CHUNK, CHUNK)],
            o_sems[b])
        if g + 2 < NUM_CHUNKS:
            in_flight[g + 2] = start_in(g + 2)

    out_flight.pop(NUM_CHUNKS - 2).wait()
    out_flight.pop(NUM_CHUNKS - 1).wait()


@jax.jit
def _run(inputs, table):
    xyq = jnp.asarray(_XYQ)
    b = jnp.asarray(_B)
    bswap = jnp.asarray(_BSWAP)
    return _sc_embed_pe(inputs.astype(jnp.int32), xyq, b, bswap, table)


def kernel(inputs, table):
    return _run(inputs, table)


# speedup vs baseline: 1.4362x; 1.1740x over previous
import functools
import math

import numpy as np
import jax
import jax.numpy as jnp
from jax import lax
from jax.experimental import pallas as pl
from jax.experimental.pallas import tpu as pltpu
from jax.experimental.pallas import tpu_sc as plsc

D_MODEL = 512
SEQ_LEN = 8192
SCALE = math.sqrt(D_MODEL)

NUM_CORES = 2
NUM_SUBCORES = 16
NUM_WORKERS = NUM_CORES * NUM_SUBCORES
ROWS_PER_WORKER = SEQ_LEN // NUM_WORKERS
CHUNK = 32
NUM_CHUNKS = ROWS_PER_WORKER // CHUNK
LANES = 16


def _host_tables():
    half = np.arange(0, D_MODEL, 2, dtype=np.float32).astype(np.float64)
    w = np.exp(half * (-math.log(10000.0) / D_MODEL))

    def pe_row(s):
        row = np.empty((D_MODEL,), dtype=np.float64)
        row[0::2] = np.sin(s * w)
        row[1::2] = np.cos(s * w)
        return row

    b = np.stack([pe_row(k * ROWS_PER_WORKER) for k in range(NUM_WORKERS)])
    bswap = np.empty_like(b)
    bswap[:, 0::2] = b[:, 1::2]
    bswap[:, 1::2] = b[:, 0::2]

    r = np.arange(ROWS_PER_WORKER, dtype=np.float64)[:, None]
    cosr = np.cos(r * w[None, :])
    sinr = np.sin(r * w[None, :])
    xq = np.empty((ROWS_PER_WORKER, D_MODEL), dtype=np.float32)
    yq = np.empty((ROWS_PER_WORKER, D_MODEL), dtype=np.float32)
    xq[:, 0::2] = cosr
    xq[:, 1::2] = cosr
    yq[:, 0::2] = sinr
    yq[:, 1::2] = -sinr

    def bf16_bits(x):
        u = x.astype(np.float32).view(np.uint32)
        rnd = ((u >> 16) & 1) + 0x7FFF
        return ((u + rnd) >> 16).astype(np.uint32)

    packed = (bf16_bits(yq) << 16) | bf16_bits(xq)
    return (
        packed.astype(np.int32),
        b.astype(np.float32),
        bswap.astype(np.float32),
    )


_XYQ, _B, _BSWAP = _host_tables()

_MESH = plsc.VectorSubcoreMesh(core_axis_name="c", subcore_axis_name="s")


@functools.partial(
    pl.kernel,
    mesh=_MESH,
    out_type=jax.ShapeDtypeStruct((SEQ_LEN, D_MODEL), jnp.float32),
    scratch_types=[
        pltpu.VMEM((ROWS_PER_WORKER,), jnp.int32),
        pltpu.VMEM((2, CHUNK, D_MODEL), jnp.float32),
        pltpu.VMEM((2, CHUNK, D_MODEL), jnp.float32),
        pltpu.VMEM((2, CHUNK, D_MODEL), jnp.int32),
        pltpu.VMEM((NUM_SUBCORES, D_MODEL), jnp.int32),
        pltpu.VMEM((1, D_MODEL), jnp.float32),
        pltpu.VMEM((1, D_MODEL), jnp.float32),
        pltpu.VMEM_SHARED((ROWS_PER_WORKER, D_MODEL), jnp.int32),
        pltpu.SemaphoreType.DMA,
        pltpu.SemaphoreType.DMA,
        pltpu.SemaphoreType.DMA,
        pltpu.SemaphoreType.DMA,
        pltpu.SemaphoreType.DMA,
        pltpu.SemaphoreType.DMA,
    ],
)
def _sc_embed_pe(idx_hbm, xyq_hbm, b_hbm, bswap_hbm, table_hbm, out_hbm,
                 idx_v, rows_v, out_v, xy_v, stage_v, b_v, bs_v, xyq_sh,
                 g_sem0, g_sem1, q_sem0, q_sem1, o_sem0, o_sem1):
    cid = lax.axis_index("c")
    sid = lax.axis_index("s")
    wid = sid * NUM_CORES + cid
    base = wid * ROWS_PER_WORKER

    g_sems = (g_sem0, g_sem1)
    q_sems = (q_sem0, q_sem1)
    o_sems = (o_sem0, o_sem1)

    pltpu.sync_copy(xyq_hbm.at[pl.ds(sid * NUM_SUBCORES, NUM_SUBCORES)], stage_v)
    pltpu.sync_copy(stage_v, xyq_sh.at[pl.ds(sid * NUM_SUBCORES, NUM_SUBCORES)])

    pltpu.sync_copy(idx_hbm.at[pl.ds(base, ROWS_PER_WORKER)], idx_v)
    pltpu.sync_copy(b_hbm.at[pl.ds(wid, 1)], b_v)
    pltpu.sync_copy(bswap_hbm.at[pl.ds(wid, 1)], bs_v)

    plsc.subcore_barrier()

    def start_in(g, b):
        pltpu.async_copy(
            table_hbm.at[idx_v.at[pl.ds(g * CHUNK, CHUNK)]],
            rows_v.at[b], g_sems[b])
        pltpu.async_copy(
            xyq_sh.at[pl.ds(g * CHUNK, CHUNK)], xy_v.at[b], q_sems[b])

    def wait_in(b):
        dummy = out_hbm.at[pl.ds(0, CHUNK)]
        pltpu.make_async_copy(dummy, rows_v.at[b], g_sems[b]).wait()
        pltpu.make_async_copy(dummy, xy_v.at[b], q_sems[b]).wait()

    def wait_out(b):
        pltpu.make_async_copy(out_hbm.at[pl.ds(0, CHUNK)], out_v.at[b],
                              o_sems[b]).wait()

    def compute(b):
        @plsc.parallel_loop(0, D_MODEL // LANES, unroll=2)
        def col_body(c):
            sl = pl.ds(c * LANES, LANES)
            bvec = b_v[0, sl]
            bsvec = bs_v[0, sl]
            for r in range(CHUNK):
                w = xy_v[b, r, sl]
                xq = lax.bitcast_convert_type(w << 16, jnp.float32)
                yq = lax.bitcast_convert_type(w, jnp.float32)
                rv = rows_v[b, r, sl]
                out_v[b, r, sl] = rv * SCALE + (bvec * xq + bsvec * yq)

    start_in(0, 0)
    start_in(1, 1)

    def pair_body(p, _):
        for b in (0, 1):
            g = 2 * p + b
            wait_in(b)

            @pl.when(p > 0)
            def _():
                wait_out(b)

            compute(b)
            pltpu.async_copy(
                out_v.at[b], out_hbm.at[pl.ds(base + g * CHUNK, CHUNK)],
                o_sems[b])

            @pl.when(p < NUM_CHUNKS // 2 - 1)
            def _():
                start_in(g + 2, b)

        return 0

    lax.fori_loop(0, NUM_CHUNKS // 2, pair_body, 0)
    wait_out(0)
    wait_out(1)


@jax.jit
def _run(inputs, table):
    xyq = jnp.asarray(_XYQ)
    b = jnp.asarray(_B)
    bswap = jnp.asarray(_BSWAP)
    return _sc_embed_pe(inputs.astype(jnp.int32), xyq, b, bswap, table)


def kernel(inputs, table):
    return _run(inputs, table)


# speedup vs baseline: 1.4935x; 1.0399x over previous
import functools
import math

import numpy as np
import jax
import jax.numpy as jnp
from jax import lax
from jax.experimental import pallas as pl
from jax.experimental.pallas import tpu as pltpu
from jax.experimental.pallas import tpu_sc as plsc

D_MODEL = 512
SEQ_LEN = 8192
SCALE = math.sqrt(D_MODEL)

NUM_CORES = 2
NUM_SUBCORES = 16
NUM_WORKERS = NUM_CORES * NUM_SUBCORES
ROWS_PER_WORKER = SEQ_LEN // NUM_WORKERS
CHUNK = 32
NUM_CHUNKS = ROWS_PER_WORKER // CHUNK
LANES = 16


def _host_tables():
    half = np.arange(0, D_MODEL, 2, dtype=np.float32).astype(np.float64)
    w = np.exp(half * (-math.log(10000.0) / D_MODEL))

    def pe_row(s):
        row = np.empty((D_MODEL,), dtype=np.float64)
        row[0::2] = np.sin(s * w)
        row[1::2] = np.cos(s * w)
        return row

    b = np.stack([pe_row(k * ROWS_PER_WORKER) for k in range(NUM_WORKERS)])
    bswap = np.empty_like(b)
    bswap[:, 0::2] = b[:, 1::2]
    bswap[:, 1::2] = b[:, 0::2]

    r = np.arange(ROWS_PER_WORKER, dtype=np.float64)[:, None]
    cosr = np.cos(r * w[None, :])
    sinr = np.sin(r * w[None, :])
    xq = np.empty((ROWS_PER_WORKER, D_MODEL), dtype=np.float32)
    yq = np.empty((ROWS_PER_WORKER, D_MODEL), dtype=np.float32)
    xq[:, 0::2] = cosr
    xq[:, 1::2] = cosr
    yq[:, 0::2] = sinr
    yq[:, 1::2] = -sinr

    def bf16_bits(x):
        u = x.astype(np.float32).view(np.uint32)
        rnd = ((u >> 16) & 1) + 0x7FFF
        return ((u + rnd) >> 16).astype(np.uint32)

    packed = (bf16_bits(yq) << 16) | bf16_bits(xq)
    return (
        packed.astype(np.int32),
        b.astype(np.float32),
        bswap.astype(np.float32),
    )


_XYQ, _B, _BSWAP = _host_tables()

_MESH = plsc.VectorSubcoreMesh(core_axis_name="c", subcore_axis_name="s")


@functools.partial(
    pl.kernel,
    mesh=_MESH,
    out_type=jax.ShapeDtypeStruct((SEQ_LEN, D_MODEL), jnp.float32),
    scratch_types=[
        pltpu.VMEM((ROWS_PER_WORKER,), jnp.int32),
        pltpu.VMEM((2, CHUNK, D_MODEL), jnp.float32),
        pltpu.VMEM((2, CHUNK, D_MODEL), jnp.float32),
        pltpu.VMEM((2, CHUNK, D_MODEL), jnp.int32),
        pltpu.VMEM((NUM_SUBCORES, D_MODEL), jnp.int32),
        pltpu.VMEM((1, D_MODEL), jnp.float32),
        pltpu.VMEM((1, D_MODEL), jnp.float32),
        pltpu.VMEM_SHARED((ROWS_PER_WORKER, D_MODEL), jnp.int32),
        pltpu.SemaphoreType.DMA,
        pltpu.SemaphoreType.DMA,
        pltpu.SemaphoreType.DMA,
        pltpu.SemaphoreType.DMA,
        pltpu.SemaphoreType.DMA,
        pltpu.SemaphoreType.DMA,
    ],
)
def _sc_embed_pe(idx_hbm, xyq_hbm, b_hbm, bswap_hbm, table_hbm, out_hbm,
                 idx_v, rows_v, out_v, xy_v, stage_v, b_v, bs_v, xyq_sh,
                 g_sem0, g_sem1, q_sem0, q_sem1, o_sem0, o_sem1):
    cid = lax.axis_index("c")
    sid = lax.axis_index("s")
    wid = sid * NUM_CORES + cid
    base = wid * ROWS_PER_WORKER

    g_sems = (g_sem0, g_sem1)
    q_sems = (q_sem0, q_sem1)
    o_sems = (o_sem0, o_sem1)

    pltpu.sync_copy(xyq_hbm.at[pl.ds(sid * NUM_SUBCORES, NUM_SUBCORES)], stage_v)
    pltpu.sync_copy(stage_v, xyq_sh.at[pl.ds(sid * NUM_SUBCORES, NUM_SUBCORES)])

    pltpu.sync_copy(idx_hbm.at[pl.ds(base, ROWS_PER_WORKER)], idx_v)
    pltpu.sync_copy(b_hbm.at[pl.ds(wid, 1)], b_v)
    pltpu.sync_copy(bswap_hbm.at[pl.ds(wid, 1)], bs_v)

    plsc.subcore_barrier()

    def start_in(g, b):
        pltpu.async_copy(
            table_hbm.at[idx_v.at[pl.ds(g * CHUNK, CHUNK)]],
            rows_v.at[b], g_sems[b])
        pltpu.async_copy(
            xyq_sh.at[pl.ds(g * CHUNK, CHUNK)], xy_v.at[b], q_sems[b])

    def wait_in(b):
        dummy = out_hbm.at[pl.ds(0, CHUNK)]
        pltpu.make_async_copy(dummy, rows_v.at[b], g_sems[b]).wait()
        pltpu.make_async_copy(dummy, xy_v.at[b], q_sems[b]).wait()

    def wait_out(b):
        pltpu.make_async_copy(out_hbm.at[pl.ds(0, CHUNK)], out_v.at[b],
                              o_sems[b]).wait()

    def compute(b):
        @plsc.parallel_loop(0, D_MODEL // LANES, unroll=4)
        def col_body(c):
            sl = pl.ds(c * LANES, LANES)
            bvec = b_v[0, sl]
            bsvec = bs_v[0, sl]
            for r in range(CHUNK):
                w = xy_v[b, r, sl]
                xq = lax.bitcast_convert_type(w << 16, jnp.float32)
                yq = lax.bitcast_convert_type(w, jnp.float32)
                rv = rows_v[b, r, sl]
                out_v[b, r, sl] = rv * SCALE + (bvec * xq + bsvec * yq)

    start_in(0, 0)
    start_in(1, 1)

    def pair_body(p, _):
        for b in (0, 1):
            g = 2 * p + b
            wait_in(b)

            @pl.when(p > 0)
            def _():
                wait_out(b)

            compute(b)
            pltpu.async_copy(
                out_v.at[b], out_hbm.at[pl.ds(base + g * CHUNK, CHUNK)],
                o_sems[b])

            @pl.when(p < NUM_CHUNKS // 2 - 1)
            def _():
                start_in(g + 2, b)

        return 0

    lax.fori_loop(0, NUM_CHUNKS // 2, pair_body, 0)
    wait_out(0)
    wait_out(1)


@jax.jit
def _run(inputs, table):
    xyq = jnp.asarray(_XYQ)
    b = jnp.asarray(_B)
    bswap = jnp.asarray(_BSWAP)
    return _sc_embed_pe(inputs.astype(jnp.int32), xyq, b, bswap, table)


def kernel(inputs, table):
    return _run(inputs, table)


# speedup vs baseline: 1.5179x; 1.0163x over previous
import functools
import math

import numpy as np
import jax
import jax.numpy as jnp
from jax import lax
from jax.experimental import pallas as pl
from jax.experimental.pallas import tpu as pltpu
from jax.experimental.pallas import tpu_sc as plsc

D_MODEL = 512
SEQ_LEN = 8192
SCALE = math.sqrt(D_MODEL)

NUM_CORES = 2
NUM_SUBCORES = 16
NUM_WORKERS = NUM_CORES * NUM_SUBCORES
ROWS_PER_WORKER = SEQ_LEN // NUM_WORKERS
CHUNK = 32
NUM_CHUNKS = ROWS_PER_WORKER // CHUNK
LANES = 16


def _host_tables():
    half = np.arange(0, D_MODEL, 2, dtype=np.float32).astype(np.float64)
    w = np.exp(half * (-math.log(10000.0) / D_MODEL))

    def pe_row(s):
        row = np.empty((D_MODEL,), dtype=np.float64)
        row[0::2] = np.sin(s * w)
        row[1::2] = np.cos(s * w)
        return row

    b = np.stack([pe_row(k * ROWS_PER_WORKER) for k in range(NUM_WORKERS)])
    bswap = np.empty_like(b)
    bswap[:, 0::2] = b[:, 1::2]
    bswap[:, 1::2] = b[:, 0::2]

    r = np.arange(ROWS_PER_WORKER, dtype=np.float64)[:, None]
    cosr = np.cos(r * w[None, :])
    sinr = np.sin(r * w[None, :])
    xq = np.empty((ROWS_PER_WORKER, D_MODEL), dtype=np.float32)
    yq = np.empty((ROWS_PER_WORKER, D_MODEL), dtype=np.float32)
    xq[:, 0::2] = cosr
    xq[:, 1::2] = cosr
    yq[:, 0::2] = sinr
    yq[:, 1::2] = -sinr

    def bf16_bits(x):
        u = x.astype(np.float32).view(np.uint32)
        rnd = ((u >> 16) & 1) + 0x7FFF
        return ((u + rnd) >> 16).astype(np.uint32)

    packed = (bf16_bits(yq) << 16) | bf16_bits(xq)
    return (
        packed.astype(np.int32),
        b.astype(np.float32),
        bswap.astype(np.float32),
    )


_XYQ, _B, _BSWAP = _host_tables()

_MESH = plsc.VectorSubcoreMesh(core_axis_name="c", subcore_axis_name="s")


@functools.partial(
    pl.kernel,
    mesh=_MESH,
    out_type=jax.ShapeDtypeStruct((SEQ_LEN, D_MODEL), jnp.float32),
    scratch_types=[
        pltpu.VMEM((ROWS_PER_WORKER,), jnp.int32),
        pltpu.VMEM((2, CHUNK, D_MODEL), jnp.float32),
        pltpu.VMEM((2, CHUNK, D_MODEL), jnp.float32),
        pltpu.VMEM((2, CHUNK, D_MODEL), jnp.int32),
        pltpu.VMEM((NUM_SUBCORES, D_MODEL), jnp.int32),
        pltpu.VMEM((1, D_MODEL), jnp.float32),
        pltpu.VMEM((1, D_MODEL), jnp.float32),
        pltpu.VMEM_SHARED((ROWS_PER_WORKER, D_MODEL), jnp.int32),
        pltpu.SemaphoreType.DMA,
        pltpu.SemaphoreType.DMA,
        pltpu.SemaphoreType.DMA,
        pltpu.SemaphoreType.DMA,
        pltpu.SemaphoreType.DMA,
        pltpu.SemaphoreType.DMA,
    ],
)
def _sc_embed_pe(idx_hbm, xyq_hbm, b_hbm, bswap_hbm, table_hbm, out_hbm,
                 idx_v, rows_v, out_v, xy_v, stage_v, b_v, bs_v, xyq_sh,
                 g_sem0, g_sem1, q_sem0, q_sem1, o_sem0, o_sem1):
    cid = lax.axis_index("c")
    sid = lax.axis_index("s")
    wid = sid * NUM_CORES + cid
    base = wid * ROWS_PER_WORKER

    g_sems = (g_sem0, g_sem1)
    q_sems = (q_sem0, q_sem1)
    o_sems = (o_sem0, o_sem1)

    pltpu.sync_copy(xyq_hbm.at[pl.ds(sid * NUM_SUBCORES, NUM_SUBCORES)], stage_v)
    pltpu.sync_copy(stage_v, xyq_sh.at[pl.ds(sid * NUM_SUBCORES, NUM_SUBCORES)])

    pltpu.sync_copy(idx_hbm.at[pl.ds(base, ROWS_PER_WORKER)], idx_v)
    pltpu.sync_copy(b_hbm.at[pl.ds(wid, 1)], b_v)
    pltpu.sync_copy(bswap_hbm.at[pl.ds(wid, 1)], bs_v)

    plsc.subcore_barrier()

    def start_in(g, b):
        pltpu.async_copy(
            table_hbm.at[idx_v.at[pl.ds(g * CHUNK, CHUNK)]],
            rows_v.at[b], g_sems[b])
        pltpu.async_copy(
            xyq_sh.at[pl.ds(g * CHUNK, CHUNK)], xy_v.at[b], q_sems[b])

    def wait_in(b):
        dummy = out_hbm.at[pl.ds(0, CHUNK)]
        pltpu.make_async_copy(dummy, rows_v.at[b], g_sems[b]).wait()
        pltpu.make_async_copy(dummy, xy_v.at[b], q_sems[b]).wait()

    def wait_out(b):
        pltpu.make_async_copy(out_hbm.at[pl.ds(0, CHUNK)], out_v.at[b],
                              o_sems[b]).wait()

    def compute(b):
        @plsc.parallel_loop(0, D_MODEL // LANES, unroll=8)
        def col_body(c):
            sl = pl.ds(c * LANES, LANES)
            bvec = b_v[0, sl]
            bsvec = bs_v[0, sl]
            for r in range(CHUNK):
                w = xy_v[b, r, sl]
                xq = lax.bitcast_convert_type(w << 16, jnp.float32)
                yq = lax.bitcast_convert_type(w, jnp.float32)
                rv = rows_v[b, r, sl]
                out_v[b, r, sl] = rv * SCALE + (bvec * xq + bsvec * yq)

    start_in(0, 0)
    start_in(1, 1)

    def pair_body(p, _):
        for b in (0, 1):
            g = 2 * p + b
            wait_in(b)

            @pl.when(p > 0)
            def _():
                wait_out(b)

            compute(b)
            pltpu.async_copy(
                out_v.at[b], out_hbm.at[pl.ds(base + g * CHUNK, CHUNK)],
                o_sems[b])

            @pl.when(p < NUM_CHUNKS // 2 - 1)
            def _():
                start_in(g + 2, b)

        return 0

    lax.fori_loop(0, NUM_CHUNKS // 2, pair_body, 0)
    wait_out(0)
    wait_out(1)


@jax.jit
def _run(inputs, table):
    xyq = jnp.asarray(_XYQ)
    b = jnp.asarray(_B)
    bswap = jnp.asarray(_BSWAP)
    return _sc_embed_pe(inputs.astype(jnp.int32), xyq, b, bswap, table)


def kernel(inputs, table):
    return _run(inputs, table)
